# baseline (speedup 1.0000x reference)
import functools

import jax
import jax.numpy as jnp
from jax import lax
from jax.experimental import pallas as pl
from jax.experimental.pallas import tpu as pltpu
from jax.experimental.pallas import tpu_sc as plsc

D = 128
DW = D // 2
PAD = 0
T = 20
NC, NS, L = 2, 16, 16
NW = NC * NS
WPR = DW // L
VPR = D // L

NODES_PER_CHUNK = 4
ROWS_PER_CHUNK = NODES_PER_CHUNK * T
RD = 4

PACK_CHUNK = 125


def _pack_body(rows_per_w, table_hbm, packed_hbm,
               fb_a, fb_b, pb_a, pb_b, isem_a, isem_b, osem_a, osem_b):
    wid = lax.axis_index("s") * NC + lax.axis_index("c")
    base = wid * rows_per_w
    n_chunks = rows_per_w // PACK_CHUNK

    def src(c):
        return table_hbm.at[pl.ds(base + c * PACK_CHUNK, PACK_CHUNK)]

    def dst(c):
        return packed_hbm.at[pl.ds(base + c * PACK_CHUNK, PACK_CHUNK)]

    def convert(fb, pb):
        @pl.loop(0, PACK_CHUNK)
        def _row(r):
            for q in range(WPR):
                a = fb[r, pl.ds(q * 2 * L, L)]
                b = fb[r, pl.ds(q * 2 * L + L, L)]
                p = plsc.pack(a, b, format=plsc.PackFormat.INTERLEAVED)
                pb[r, pl.ds(q * L, L)] = plsc.bitcast(p, jnp.int32)

    pltpu.async_copy(src(0), fb_a, isem_a)
    pltpu.async_copy(src(1), fb_b, isem_b)

    def do_chunk(c, fb, pb, isem, osem):
        pltpu.make_async_copy(src(c), fb, isem).wait()
        @pl.when(c >= 2)
        def _():
            pltpu.make_async_copy(pb, dst(c - 2), osem).wait()
        convert(fb, pb)
        @pl.when(c + 2 < n_chunks)
        def _():
            pltpu.async_copy(src(c + 2), fb, isem)
        pltpu.async_copy(pb, dst(c), osem)

    @pl.loop(0, n_chunks - 1, step=2)
    def _chunk(c):
        do_chunk(c, fb_a, pb_a, isem_a, osem_a)
        do_chunk(c + 1, fb_b, pb_b, isem_b, osem_b)

    do_chunk(n_chunks - 1, fb_a, pb_a, isem_a, osem_a)

    pltpu.make_async_copy(pb_a, dst(n_chunks - 1), osem_a).wait()
    pltpu.make_async_copy(pb_b, dst(n_chunks - 2), osem_b).wait()


def _compute_chunk(idx_v, rows_v, out_v, idx_off):
    @pl.loop(0, NODES_PER_CHUNK)
    def _node(n):
        base = idx_off + n * T
        iv1 = idx_v[pl.ds(base, L)]
        iv2 = idx_v[pl.ds(base + T - L, L)]
        m1 = jnp.where(iv1 != PAD, jnp.int32(-1), jnp.int32(0))
        m2 = jnp.where(iv2 != PAD, jnp.int32(-1), jnp.int32(0))
        acc = [None] * WPR
        for t in range(T):
            r = n * T + t
            m = m1[t] if t < L else m2[t - (T - L)]
            for q in range(WPR):
                w = rows_v[r, pl.ds(q * L, L)] & m
                v = plsc.bitcast(w, jnp.bfloat16)
                acc[q] = v if t == 0 else jnp.maximum(acc[q], v)
        for q in range(WPR):
            a, b = plsc.unpack(acc[q], format=plsc.PackFormat.INTERLEAVED)
            out_v[n, pl.ds(q * 2 * L, L)] = a
            out_v[n, pl.ds(q * 2 * L + L, L)] = b


def _gather_body(rows_per_w, nodes_per_w,
                 xf_hbm, packed_hbm, out_hbm, idx_v, rows_bufs, out_bufs,
                 gsems, osems):
    wid = lax.axis_index("s") * NC + lax.axis_index("c")
    row_base = wid * rows_per_w
    node_base = wid * nodes_per_w
    pltpu.sync_copy(xf_hbm.at[pl.ds(row_base, rows_per_w)], idx_v)
    n_chunks = nodes_per_w // NODES_PER_CHUNK

    def gather_src(c):
        return packed_hbm.at[idx_v.at[pl.ds(c * ROWS_PER_CHUNK,
                                            ROWS_PER_CHUNK)]]

    def out_dst(c):
        return out_hbm.at[pl.ds(node_base + c * NODES_PER_CHUNK,
                                NODES_PER_CHUNK)]

    for k in range(RD):
        pltpu.async_copy(gather_src(k), rows_bufs[k], gsems[k])

    def do_chunk(c, k):
        rows_v, out_v, gsem, osem = (rows_bufs[k], out_bufs[k],
                                     gsems[k], osems[k])
        pltpu.make_async_copy(gather_src(c), rows_v, gsem).wait()
        @pl.when(c >= RD)
        def _():
            pltpu.make_async_copy(out_v, out_dst(c - RD), osem).wait()
        _compute_chunk(idx_v, rows_v, out_v, c * ROWS_PER_CHUNK)
        @pl.when(c + RD < n_chunks)
        def _():
            pltpu.async_copy(gather_src(c + RD), rows_v, gsem)
        pltpu.async_copy(out_v, out_dst(c), osem)

    @pl.loop(0, n_chunks, step=RD)
    def _chunk(c):
        for k in range(RD):
            do_chunk(c + k, k)

    for k in range(RD):
        pltpu.make_async_copy(
            out_bufs[k], out_dst(n_chunks - RD + k), osems[k]).wait()


def _gather_flat(rows_per_w, nodes_per_w, xf_hbm, packed_hbm, out_hbm,
                 idx_v, *bufs):
    rows_bufs = list(bufs[0:RD])
    out_bufs = list(bufs[RD:2 * RD])
    gsems = list(bufs[2 * RD:3 * RD])
    osems = list(bufs[3 * RD:4 * RD])
    _gather_body(rows_per_w, nodes_per_w, xf_hbm, packed_hbm, out_hbm,
                 idx_v, rows_bufs, out_bufs, gsems, osems)


@jax.jit
def kernel(x, table):
    B, N, Tk = x.shape
    V = table.shape[0]
    assert Tk == T and table.shape[1] == D
    nodes = B * N
    rows = nodes * T
    rows_per_w = rows // NW
    nodes_per_w = nodes // NW
    vrows_per_w = V // NW
    xf = x.reshape(rows)
    mesh = plsc.VectorSubcoreMesh(core_axis_name="c", subcore_axis_name="s")
    params = pltpu.CompilerParams(needs_layout_passes=False,
                                  use_tc_tiling_on_sc=False)

    packed = pl.kernel(
        functools.partial(_pack_body, vrows_per_w),
        out_type=jax.ShapeDtypeStruct((V, DW), jnp.int32),
        mesh=mesh,
        scratch_types=(
            [pltpu.VMEM((PACK_CHUNK, D), jnp.float32)] * 2
            + [pltpu.VMEM((PACK_CHUNK, DW), jnp.int32)] * 2
            + [pltpu.SemaphoreType.DMA] * 4),
        compiler_params=params,
    )(table)

    scratch = ([pltpu.VMEM((rows_per_w,), jnp.int32)]
               + [pltpu.VMEM((ROWS_PER_CHUNK, DW), jnp.int32)] * RD
               + [pltpu.VMEM((NODES_PER_CHUNK, D), jnp.float32)] * RD
               + [pltpu.SemaphoreType.DMA] * (2 * RD))
    out = pl.kernel(
        functools.partial(_gather_flat, rows_per_w, nodes_per_w),
        out_type=jax.ShapeDtypeStruct((nodes, D), jnp.float32),
        mesh=mesh,
        scratch_types=scratch,
        compiler_params=params,
    )(xf, packed)
    return out.reshape(B, N, D)


# speedup vs baseline: 1.1624x; 1.1624x over previous
import functools

import jax
import jax.numpy as jnp
from jax import lax
from jax.experimental import pallas as pl
from jax.experimental.pallas import tpu as pltpu
from jax.experimental.pallas import tpu_sc as plsc

D = 128
PAD = 0
T = 20
NC, NS, L = 2, 16, 16
NW = NC * NS
VPR = D // L

NODES_PER_CHUNK = 8
ROWS_PER_CHUNK = NODES_PER_CHUNK * T
HALF_ROWS = ROWS_PER_CHUNK // 2
RD = 4


def _compute_chunk(idx_v, rows_v, out_v, idx_off):
    @pl.loop(0, NODES_PER_CHUNK)
    def _node(n):
        base = idx_off + n * T
        iv1 = idx_v[pl.ds(base, L)]
        iv2 = idx_v[pl.ds(base + T - L, L)]
        m1 = jnp.where(iv1 != PAD, jnp.float32(1.0), jnp.float32(0.0))
        m2 = jnp.where(iv2 != PAD, jnp.float32(1.0), jnp.float32(0.0))
        acc = [None] * VPR
        for t in range(T):
            r = n * T + t
            m = m1[t] if t < L else m2[t - (T - L)]
            for q in range(VPR):
                v = rows_v[r, pl.ds(q * L, L)] * m
                acc[q] = v if t == 0 else jnp.maximum(acc[q], v)
        for q in range(VPR):
            out_v[n, pl.ds(q * L, L)] = acc[q]


def _body(rows_per_w, nodes_per_w,
          xf_hbm, table_hbm, out_hbm, idx_v, rows_bufs, out_bufs,
          gsems, osems):
    wid = lax.axis_index("s") * NC + lax.axis_index("c")
    row_base = wid * rows_per_w
    node_base = wid * nodes_per_w
    pltpu.sync_copy(xf_hbm.at[pl.ds(row_base, rows_per_w)], idx_v)
    n_chunks = nodes_per_w // NODES_PER_CHUNK

    def gather_src(c, h):
        return table_hbm.at[idx_v.at[pl.ds(c * ROWS_PER_CHUNK + h * HALF_ROWS,
                                           HALF_ROWS)]]

    def start_gather(c, rows_v, gsem):
        pltpu.async_copy(gather_src(c, 0), rows_v.at[pl.ds(0, HALF_ROWS)],
                         gsem)
        pltpu.async_copy(gather_src(c, 1),
                         rows_v.at[pl.ds(HALF_ROWS, HALF_ROWS)], gsem)

    def wait_gather(c, rows_v, gsem):
        pltpu.make_async_copy(gather_src(c, 0),
                              rows_v.at[pl.ds(0, HALF_ROWS)], gsem).wait()
        pltpu.make_async_copy(gather_src(c, 1),
                              rows_v.at[pl.ds(HALF_ROWS, HALF_ROWS)],
                              gsem).wait()

    def out_dst(c):
        return out_hbm.at[pl.ds(node_base + c * NODES_PER_CHUNK,
                                NODES_PER_CHUNK)]

    for k in range(RD):
        start_gather(k, rows_bufs[k], gsems[k])

    def do_chunk(c, k):
        rows_v, out_v, gsem, osem = rows_bufs[k], out_bufs[k], gsems[k], osems[k]
        wait_gather(c, rows_v, gsem)
        @pl.when(c >= RD)
        def _():
            pltpu.make_async_copy(out_v, out_dst(c - RD), osem).wait()
        _compute_chunk(idx_v, rows_v, out_v, c * ROWS_PER_CHUNK)
        @pl.when(c + RD < n_chunks)
        def _():
            start_gather(c + RD, rows_v, gsem)
        pltpu.async_copy(out_v, out_dst(c), osem)

    @pl.loop(0, n_chunks, step=RD)
    def _chunk(c):
        for k in range(RD):
            do_chunk(c + k, k)

    for k in range(RD):
        pltpu.make_async_copy(
            out_bufs[k], out_dst(n_chunks - RD + k), osems[k]).wait()


def _body_flat(rows_per_w, nodes_per_w, xf_hbm, table_hbm, out_hbm,
               idx_v, *bufs):
    rows_bufs = list(bufs[0:RD])
    out_bufs = list(bufs[RD:2 * RD])
    gsems = list(bufs[2 * RD:3 * RD])
    osems = list(bufs[3 * RD:4 * RD])
    _body(rows_per_w, nodes_per_w, xf_hbm, table_hbm, out_hbm,
          idx_v, rows_bufs, out_bufs, gsems, osems)


@jax.jit
def kernel(x, table):
    B, N, Tk = x.shape
    assert Tk == T and table.shape[1] == D
    nodes = B * N
    rows = nodes * T
    rows_per_w = rows // NW
    nodes_per_w = nodes // NW
    xf = x.reshape(rows)
    mesh = plsc.VectorSubcoreMesh(core_axis_name="c", subcore_axis_name="s")
    scratch = ([pltpu.VMEM((rows_per_w,), jnp.int32)]
               + [pltpu.VMEM((ROWS_PER_CHUNK, D), jnp.float32)] * RD
               + [pltpu.VMEM((NODES_PER_CHUNK, D), jnp.float32)] * RD
               + [pltpu.SemaphoreType.DMA] * (2 * RD))
    out = pl.kernel(
        functools.partial(_body_flat, rows_per_w, nodes_per_w),
        out_type=jax.ShapeDtypeStruct((nodes, D), jnp.float32),
        mesh=mesh,
        scratch_types=scratch,
        compiler_params=pltpu.CompilerParams(needs_layout_passes=False,
                                             use_tc_tiling_on_sc=False),
    )(xf, table)
    return out.reshape(B, N, D)
